# baseline (speedup 1.0000x reference)
import functools

import jax
import jax.numpy as jnp
from jax import lax
from jax.experimental import pallas as pl
from jax.experimental.pallas import tpu as pltpu
from jax.experimental.pallas import tpu_sc as plsc


def _rne16(u):
    return (u + jnp.uint32(0x7FFF) + ((u >> jnp.uint32(16)) & jnp.uint32(1))
            ) >> jnp.uint32(16)


def _pack_pair(x, HP):
    a = lax.bitcast_convert_type(x[:, :HP], jnp.uint32)
    b = lax.bitcast_convert_type(x[:, HP:], jnp.uint32)
    w = (_rne16(a) & jnp.uint32(0xFFFF)) | (_rne16(b) << jnp.uint32(16))
    return lax.bitcast_convert_type(w, jnp.float32)


def _pack_body(HP, h_ref, c_ref, t_ref):
    t_ref[:, :HP] = _pack_pair(h_ref[...], HP)
    t_ref[:, HP:] = _pack_pair(c_ref[...], HP)


def _tc_pack(h, c, Mp=2000):
    n, HH = h.shape
    HP = HH // 2
    return pl.pallas_call(
        functools.partial(_pack_body, HP),
        grid=(n // Mp,),
        in_specs=[
            pl.BlockSpec((Mp, HH), lambda i: (i, 0)),
            pl.BlockSpec((Mp, HH), lambda i: (i, 0)),
        ],
        out_specs=pl.BlockSpec((Mp, 2 * HP), lambda i: (i, 0)),
        out_shape=jax.ShapeDtypeStruct((n, 2 * HP), jnp.float32),
    )(h, c)


def _sc_gather_body(NC, K0C, K1C, CH, HT, NBUF, DEPTH,
                    t_hbm, idx0_hbm, idx1_hbm, out,
                    i0_all, i1_all, g0_v, g1_v, *sems):
    gsems = sems[:NBUF]
    wsems = sems[NBUF:]
    cid = lax.axis_index("c")
    sid = lax.axis_index("s")
    start = jnp.where(cid == 0, sid * K0C, (16 * K0C) + sid * K1C)
    nch_w = jnp.where(cid == 0, K0C, K1C)
    wbase = start * CH

    @pl.when(cid == 0)
    def _():
        pltpu.sync_copy(idx0_hbm.at[pl.ds(wbase, K0C * CH)],
                        i0_all.at[pl.ds(0, K0C * CH)])
        pltpu.sync_copy(idx1_hbm.at[pl.ds(wbase, K0C * CH)],
                        i1_all.at[pl.ds(0, K0C * CH)])

    if K1C:
        @pl.when(cid == 1)
        def _():
            pltpu.sync_copy(idx0_hbm.at[pl.ds(wbase, K1C * CH)],
                            i0_all.at[pl.ds(0, K1C * CH)])
            pltpu.sync_copy(idx1_hbm.at[pl.ds(wbase, K1C * CH)],
                            i1_all.at[pl.ds(0, K1C * CH)])

    def g_cps(b, k):
        io0 = i0_all.at[pl.ds(k * CH, CH)]
        io1 = i1_all.at[pl.ds(k * CH, CH)]
        return (
            pltpu.make_async_copy(t_hbm.at[io0], g0_v.at[b], gsems[b]),
            pltpu.make_async_copy(t_hbm.at[io1], g1_v.at[b], gsems[b]),
        )

    def w_cps(b, k):
        rows = pl.ds(wbase + k * CH, CH)
        return (
            pltpu.make_async_copy(g0_v.at[b], out.at[rows, pl.ds(0, HT)],
                                  wsems[b]),
            pltpu.make_async_copy(g1_v.at[b], out.at[rows, pl.ds(HT, HT)],
                                  wsems[b]),
        )

    def body(it, carry):
        g = it * NBUF
        for b in range(NBUF):
            k = g + b

            @pl.when((k >= NBUF) & (k - NBUF < nch_w))
            def _():
                for cp in w_cps(b, k - NBUF):
                    cp.wait()

            @pl.when(k < nch_w)
            def _():
                for cp in g_cps(b, k):
                    cp.start()

            kd = k - DEPTH
            bd = (b - DEPTH) % NBUF

            @pl.when((kd >= 0) & (kd < nch_w))
            def _():
                for cp in g_cps(bd, kd):
                    cp.wait()
                for cp in w_cps(bd, kd):
                    cp.start()
        return carry

    kmax = max(K0C, K1C)
    lax.fori_loop(0, (kmax + 2 * NBUF - 1) // NBUF, body, 0)


def _make_sc_gather(n_pad, HT, frac0=0.5):
    info = plsc.get_sparse_core_info()
    NC, NS = info.num_cores, info.num_subcores
    NW = NC * NS
    CH = 40
    NBUF = 4
    DEPTH = 2
    assert n_pad % (NW * CH) == 0
    nch_g = n_pad // CH
    K0C = int(round(nch_g * frac0 / NS))
    K1C = nch_g // NS - K0C
    kmax = max(K0C, K1C)

    mesh = plsc.VectorSubcoreMesh(core_axis_name="c", subcore_axis_name="s")
    return functools.partial(
        pl.kernel,
        functools.partial(_sc_gather_body, NC, K0C, K1C, CH, HT, NBUF,
                          DEPTH),
        out_type=jax.ShapeDtypeStruct((n_pad, 2 * HT), jnp.float32),
        mesh=mesh,
        scratch_types=(
            [pltpu.VMEM((kmax * CH,), jnp.int32),
             pltpu.VMEM((kmax * CH,), jnp.int32)] +
            [pltpu.VMEM((NBUF, CH, HT), jnp.float32) for _ in range(2)] +
            [pltpu.SemaphoreType.DMA for _ in range(2 * NBUF)]
        ),
    )()


def _unpack(x):
    w = lax.bitcast_convert_type(x, jnp.uint32)
    lo = lax.bitcast_convert_type(w << jnp.uint32(16), jnp.float32)
    hi = lax.bitcast_convert_type(w & jnp.uint32(0xFFFF0000), jnp.float32)
    return lo, hi


def _tc_body(HH, mail_ref, wft_ref, bf_ref, wiout_ref, biou_ref,
             hnew_ref, cnew_ref):
    HP = HH // 2
    mb = mail_ref[...]
    hlo0, hhi0 = _unpack(mb[:, 0 * HP:1 * HP])
    clo0, chi0 = _unpack(mb[:, 1 * HP:2 * HP])
    hlo1, hhi1 = _unpack(mb[:, 2 * HP:3 * HP])
    clo1, chi1 = _unpack(mb[:, 3 * HP:4 * HP])
    hcat = jnp.concatenate([hlo0, hhi0, hlo1, hhi1],
                           axis=1).astype(jnp.bfloat16)
    f_pre = jnp.dot(hcat, wft_ref[...],
                    preferred_element_type=jnp.float32) + bf_ref[...]
    f = jax.nn.sigmoid(f_pre)
    cc = jnp.concatenate([clo0, chi0, clo1, chi1], axis=1)
    c_red = f[:, :HH] * cc[:, :HH] + f[:, HH:] * cc[:, HH:]
    iou = jnp.dot(hcat, wiout_ref[...],
                  preferred_element_type=jnp.float32) + biou_ref[...]
    i = jax.nn.sigmoid(iou[:, :HH])
    o = jax.nn.sigmoid(iou[:, HH:2 * HH])
    u = jnp.tanh(iou[:, 2 * HH:])
    c_new = i * u + c_red
    hnew_ref[...] = o * jnp.tanh(c_new)
    cnew_ref[...] = c_new


def _tc_compute(n, mail, wft, bf, wiout, biou, M=2000):
    fourHP = mail.shape[1]
    HH = fourHP // 2
    twoH = 2 * HH
    grid = (n // M,)
    return pl.pallas_call(
        functools.partial(_tc_body, HH),
        grid=grid,
        in_specs=[
            pl.BlockSpec((M, fourHP), lambda i: (i, 0)),
            pl.BlockSpec((twoH, twoH), lambda i: (0, 0)),
            pl.BlockSpec((1, twoH), lambda i: (0, 0)),
            pl.BlockSpec((twoH, 3 * HH), lambda i: (0, 0)),
            pl.BlockSpec((1, 3 * HH), lambda i: (0, 0)),
        ],
        out_specs=[
            pl.BlockSpec((M, HH), lambda i: (i, 0)),
            pl.BlockSpec((M, HH), lambda i: (i, 0)),
        ],
        out_shape=[
            jax.ShapeDtypeStruct((n, HH), jnp.float32),
            jax.ShapeDtypeStruct((n, HH), jnp.float32),
        ],
    )(mail, wft, bf, wiout, biou)


def kernel(h, c, child_idx, W_f, b_f, W_iou, b_iou):
    n, HH = h.shape
    HT = HH
    NW = 32
    n_per_w = -(-n // (NW * 160)) * 160
    n_pad = n_per_w * NW

    ci = child_idx.astype(jnp.int32)
    pad = jnp.zeros((n_pad - n,), jnp.int32)
    idx0 = jnp.concatenate([ci[:, 0], pad])
    idx1 = jnp.concatenate([ci[:, 1], pad])

    table = _tc_pack(h, c)
    sc_gather = _make_sc_gather(n_pad, HT, frac0=0.5)
    mail = sc_gather(table, idx0, idx1)

    h_new, c_new = _tc_compute(
        n, mail,
        W_f.T.astype(jnp.bfloat16), b_f.reshape(1, -1),
        W_iou.T.astype(jnp.bfloat16), b_iou.reshape(1, -1))
    return (h_new, c_new)


# speedup vs baseline: 1.0613x; 1.0613x over previous
import functools

import jax
import jax.numpy as jnp
from jax import lax
from jax.experimental import pallas as pl
from jax.experimental.pallas import tpu as pltpu
from jax.experimental.pallas import tpu_sc as plsc


def _rne16(u):
    return (u + jnp.uint32(0x7FFF) + ((u >> jnp.uint32(16)) & jnp.uint32(1))
            ) >> jnp.uint32(16)


def _pack_pair(x, HP):
    a = lax.bitcast_convert_type(x[:, :HP], jnp.uint32)
    b = lax.bitcast_convert_type(x[:, HP:], jnp.uint32)
    w = (_rne16(a) & jnp.uint32(0xFFFF)) | (_rne16(b) << jnp.uint32(16))
    return lax.bitcast_convert_type(w, jnp.float32)


def _pack_body(HP, h_ref, c_ref, t_ref):
    t_ref[:, :HP] = _pack_pair(h_ref[...], HP)
    t_ref[:, HP:] = _pack_pair(c_ref[...], HP)


def _tc_pack(h, c, Mp=2000):
    n, HH = h.shape
    HP = HH // 2
    return pl.pallas_call(
        functools.partial(_pack_body, HP),
        grid=(n // Mp,),
        in_specs=[
            pl.BlockSpec((Mp, HH), lambda i: (i, 0)),
            pl.BlockSpec((Mp, HH), lambda i: (i, 0)),
        ],
        out_specs=pl.BlockSpec((Mp, 2 * HP), lambda i: (i, 0)),
        out_shape=jax.ShapeDtypeStruct((n, 2 * HP), jnp.float32),
    )(h, c)


def _sc_gather_body(NC, K0C, K1C, CH, HT, NBUF, DEPTH,
                    t_hbm, idx0_hbm, idx1_hbm, out,
                    i0_all, i1_all, g0_v, g1_v, *sems):
    gsems = sems[:NBUF]
    wsems = sems[NBUF:]
    cid = lax.axis_index("c")
    sid = lax.axis_index("s")
    start = jnp.where(cid == 0, sid * K0C, (16 * K0C) + sid * K1C)
    nch_w = jnp.where(cid == 0, K0C, K1C)
    wbase = start * CH

    @pl.when(cid == 0)
    def _():
        pltpu.sync_copy(idx0_hbm.at[pl.ds(wbase, K0C * CH)],
                        i0_all.at[pl.ds(0, K0C * CH)])
        pltpu.sync_copy(idx1_hbm.at[pl.ds(wbase, K0C * CH)],
                        i1_all.at[pl.ds(0, K0C * CH)])

    if K1C:
        @pl.when(cid == 1)
        def _():
            pltpu.sync_copy(idx0_hbm.at[pl.ds(wbase, K1C * CH)],
                            i0_all.at[pl.ds(0, K1C * CH)])
            pltpu.sync_copy(idx1_hbm.at[pl.ds(wbase, K1C * CH)],
                            i1_all.at[pl.ds(0, K1C * CH)])

    def g_cps(b, k):
        io0 = i0_all.at[pl.ds(k * CH, CH)]
        io1 = i1_all.at[pl.ds(k * CH, CH)]
        return (
            pltpu.make_async_copy(t_hbm.at[io0], g0_v.at[b], gsems[b]),
            pltpu.make_async_copy(t_hbm.at[io1], g1_v.at[b], gsems[b]),
        )

    def w_cps(b, k):
        rows = pl.ds(wbase + k * CH, CH)
        return (
            pltpu.make_async_copy(g0_v.at[b], out.at[rows, pl.ds(0, HT)],
                                  wsems[b]),
            pltpu.make_async_copy(g1_v.at[b], out.at[rows, pl.ds(HT, HT)],
                                  wsems[b]),
        )

    def body(it, carry):
        g = it * NBUF
        for b in range(NBUF):
            k = g + b

            @pl.when((k >= NBUF) & (k - NBUF < nch_w))
            def _():
                for cp in w_cps(b, k - NBUF):
                    cp.wait()

            @pl.when(k < nch_w)
            def _():
                for cp in g_cps(b, k):
                    cp.start()

            kd = k - DEPTH
            bd = (b - DEPTH) % NBUF

            @pl.when((kd >= 0) & (kd < nch_w))
            def _():
                for cp in g_cps(bd, kd):
                    cp.wait()
                for cp in w_cps(bd, kd):
                    cp.start()
        return carry

    kmax = max(K0C, K1C)
    lax.fori_loop(0, (kmax + 2 * NBUF - 1) // NBUF, body, 0)


def _make_sc_gather(n_pad, HT, frac0=0.5):
    info = plsc.get_sparse_core_info()
    NC, NS = info.num_cores, info.num_subcores
    NW = NC * NS
    CH = 40
    NBUF = 4
    DEPTH = 2
    assert n_pad % (NW * CH) == 0
    nch_g = n_pad // CH
    K0C = int(round(nch_g * frac0 / NS))
    K1C = nch_g // NS - K0C
    kmax = max(K0C, K1C)

    mesh = plsc.VectorSubcoreMesh(core_axis_name="c", subcore_axis_name="s")
    return functools.partial(
        pl.kernel,
        functools.partial(_sc_gather_body, NC, K0C, K1C, CH, HT, NBUF,
                          DEPTH),
        out_type=jax.ShapeDtypeStruct((n_pad, 2 * HT), jnp.float32),
        mesh=mesh,
        scratch_types=(
            [pltpu.VMEM((kmax * CH,), jnp.int32),
             pltpu.VMEM((kmax * CH,), jnp.int32)] +
            [pltpu.VMEM((NBUF, CH, HT), jnp.float32) for _ in range(2)] +
            [pltpu.SemaphoreType.DMA for _ in range(2 * NBUF)]
        ),
    )()


def _unpack(x):
    w = lax.bitcast_convert_type(x, jnp.uint32)
    lo = lax.bitcast_convert_type(w << jnp.uint32(16), jnp.float32)
    hi = lax.bitcast_convert_type(w & jnp.uint32(0xFFFF0000), jnp.float32)
    return lo, hi


def _tc_body(HH, has_prev, *refs):
    if has_prev:
        (mail_ref, wft_ref, bf_ref, wiout_ref, biou_ref,
         _hprev, _cprev, hnew_ref, cnew_ref) = refs
    else:
        (mail_ref, wft_ref, bf_ref, wiout_ref, biou_ref,
         hnew_ref, cnew_ref) = refs
    HP = HH // 2
    mb = mail_ref[...]
    hlo0, hhi0 = _unpack(mb[:, 0 * HP:1 * HP])
    clo0, chi0 = _unpack(mb[:, 1 * HP:2 * HP])
    hlo1, hhi1 = _unpack(mb[:, 2 * HP:3 * HP])
    clo1, chi1 = _unpack(mb[:, 3 * HP:4 * HP])
    hcat = jnp.concatenate([hlo0, hhi0, hlo1, hhi1],
                           axis=1).astype(jnp.bfloat16)
    f_pre = jnp.dot(hcat, wft_ref[...],
                    preferred_element_type=jnp.float32) + bf_ref[...]
    f = jax.nn.sigmoid(f_pre)
    cc = jnp.concatenate([clo0, chi0, clo1, chi1], axis=1)
    c_red = f[:, :HH] * cc[:, :HH] + f[:, HH:] * cc[:, HH:]
    iou = jnp.dot(hcat, wiout_ref[...],
                  preferred_element_type=jnp.float32) + biou_ref[...]
    i = jax.nn.sigmoid(iou[:, :HH])
    o = jax.nn.sigmoid(iou[:, HH:2 * HH])
    u = jnp.tanh(iou[:, 2 * HH:])
    c_new = i * u + c_red
    hnew_ref[...] = o * jnp.tanh(c_new)
    cnew_ref[...] = c_new


def _tc_compute_seg(n, blk0, nblk, mail, wft, bf, wiout, biou, prev, M):
    fourHP = mail.shape[1]
    HH = fourHP // 2
    twoH = 2 * HH
    in_specs = [
        pl.BlockSpec((M, fourHP), lambda i: (i, 0)),
        pl.BlockSpec((twoH, twoH), lambda i: (0, 0)),
        pl.BlockSpec((1, twoH), lambda i: (0, 0)),
        pl.BlockSpec((twoH, 3 * HH), lambda i: (0, 0)),
        pl.BlockSpec((1, 3 * HH), lambda i: (0, 0)),
    ]
    args = [mail, wft, bf, wiout, biou]
    kwargs = {}
    if prev is not None:
        in_specs += [pl.BlockSpec(memory_space=pl.ANY),
                     pl.BlockSpec(memory_space=pl.ANY)]
        args += [prev[0], prev[1]]
        kwargs['input_output_aliases'] = {5: 0, 6: 1}
    return pl.pallas_call(
        functools.partial(_tc_body, HH, prev is not None),
        grid=(nblk,),
        in_specs=in_specs,
        out_specs=[
            pl.BlockSpec((M, HH), lambda i: (i + blk0, 0)),
            pl.BlockSpec((M, HH), lambda i: (i + blk0, 0)),
        ],
        out_shape=[
            jax.ShapeDtypeStruct((n, HH), jnp.float32),
            jax.ShapeDtypeStruct((n, HH), jnp.float32),
        ],
        **kwargs,
    )(*args)


def kernel(h, c, child_idx, W_f, b_f, W_iou, b_iou):
    n, HH = h.shape
    HT = HH
    NW = 32
    n_per_w = -(-n // (NW * 160)) * 160
    n_pad = n_per_w * NW

    ci = child_idx.astype(jnp.int32)
    pad = jnp.zeros((n_pad - n,), jnp.int32)
    idx0 = jnp.concatenate([ci[:, 0], pad])
    idx1 = jnp.concatenate([ci[:, 1], pad])

    table = _tc_pack(h, c)

    S = 2
    seg = n_pad // S
    M = 1600
    nblk = seg // M
    sc_gather = _make_sc_gather(seg, HT, frac0=0.5)
    wft = W_f.T.astype(jnp.bfloat16)
    wiout = W_iou.T.astype(jnp.bfloat16)
    bf2 = b_f.reshape(1, -1)
    biou2 = b_iou.reshape(1, -1)

    prev = None
    for s in range(S):
        mail = sc_gather(table, idx0[s * seg:(s + 1) * seg],
                         idx1[s * seg:(s + 1) * seg])
        prev = _tc_compute_seg(n, s * nblk, nblk, mail, wft, bf2, wiout,
                               biou2, prev, M)
    return (prev[0], prev[1])


# speedup vs baseline: 1.1071x; 1.0432x over previous
import functools

import jax
import jax.numpy as jnp
from jax import lax
from jax.experimental import pallas as pl
from jax.experimental.pallas import tpu as pltpu
from jax.experimental.pallas import tpu_sc as plsc


def _rne16(u):
    return (u + jnp.uint32(0x7FFF) + ((u >> jnp.uint32(16)) & jnp.uint32(1))
            ) >> jnp.uint32(16)


def _pack_pair(x, HP):
    a = lax.bitcast_convert_type(x[:, :HP], jnp.uint32)
    b = lax.bitcast_convert_type(x[:, HP:], jnp.uint32)
    w = (_rne16(a) & jnp.uint32(0xFFFF)) | (_rne16(b) << jnp.uint32(16))
    return lax.bitcast_convert_type(w, jnp.float32)


def _pack_body(HP, h_ref, c_ref, t_ref):
    t_ref[:, :HP] = _pack_pair(h_ref[...], HP)
    t_ref[:, HP:] = _pack_pair(c_ref[...], HP)


def _tc_pack(h, c, Mp=2000):
    n, HH = h.shape
    HP = HH // 2
    return pl.pallas_call(
        functools.partial(_pack_body, HP),
        grid=(n // Mp,),
        in_specs=[
            pl.BlockSpec((Mp, HH), lambda i: (i, 0)),
            pl.BlockSpec((Mp, HH), lambda i: (i, 0)),
        ],
        out_specs=pl.BlockSpec((Mp, 2 * HP), lambda i: (i, 0)),
        out_shape=jax.ShapeDtypeStruct((n, 2 * HP), jnp.float32),
    )(h, c)


def _sc_gather_body(NC, K0C, K1C, CH, HT, NBUF, DEPTH,
                    t_hbm, idx0_hbm, idx1_hbm, out,
                    i0_all, i1_all, g0_v, g1_v, *sems):
    gsems = sems[:NBUF]
    wsems = sems[NBUF:]
    cid = lax.axis_index("c")
    sid = lax.axis_index("s")
    start = jnp.where(cid == 0, sid * K0C, (16 * K0C) + sid * K1C)
    nch_w = jnp.where(cid == 0, K0C, K1C)
    wbase = start * CH

    @pl.when(cid == 0)
    def _():
        pltpu.sync_copy(idx0_hbm.at[pl.ds(wbase, K0C * CH)],
                        i0_all.at[pl.ds(0, K0C * CH)])
        pltpu.sync_copy(idx1_hbm.at[pl.ds(wbase, K0C * CH)],
                        i1_all.at[pl.ds(0, K0C * CH)])

    if K1C:
        @pl.when(cid == 1)
        def _():
            pltpu.sync_copy(idx0_hbm.at[pl.ds(wbase, K1C * CH)],
                            i0_all.at[pl.ds(0, K1C * CH)])
            pltpu.sync_copy(idx1_hbm.at[pl.ds(wbase, K1C * CH)],
                            i1_all.at[pl.ds(0, K1C * CH)])

    def g_cps(b, k):
        io0 = i0_all.at[pl.ds(k * CH, CH)]
        io1 = i1_all.at[pl.ds(k * CH, CH)]
        return (
            pltpu.make_async_copy(t_hbm.at[io0], g0_v.at[b], gsems[b]),
            pltpu.make_async_copy(t_hbm.at[io1], g1_v.at[b], gsems[b]),
        )

    def w_cps(b, k):
        rows = pl.ds(wbase + k * CH, CH)
        return (
            pltpu.make_async_copy(g0_v.at[b], out.at[rows, pl.ds(0, HT)],
                                  wsems[b]),
            pltpu.make_async_copy(g1_v.at[b], out.at[rows, pl.ds(HT, HT)],
                                  wsems[b]),
        )

    def body(it, carry):
        g = it * NBUF
        for b in range(NBUF):
            k = g + b

            @pl.when((k >= NBUF) & (k - NBUF < nch_w))
            def _():
                for cp in w_cps(b, k - NBUF):
                    cp.wait()

            @pl.when(k < nch_w)
            def _():
                for cp in g_cps(b, k):
                    cp.start()

            kd = k - DEPTH
            bd = (b - DEPTH) % NBUF

            @pl.when((kd >= 0) & (kd < nch_w))
            def _():
                for cp in g_cps(bd, kd):
                    cp.wait()
                for cp in w_cps(bd, kd):
                    cp.start()
        return carry

    kmax = max(K0C, K1C)
    lax.fori_loop(0, (kmax + 2 * NBUF - 1) // NBUF, body, 0)


def _make_sc_gather(n_pad, HT, frac0=0.5):
    info = plsc.get_sparse_core_info()
    NC, NS = info.num_cores, info.num_subcores
    NW = NC * NS
    CH = 40
    NBUF = 4
    DEPTH = 2
    assert n_pad % (NW * CH) == 0
    nch_g = n_pad // CH
    K0C = int(round(nch_g * frac0 / NS))
    K1C = nch_g // NS - K0C
    kmax = max(K0C, K1C)

    mesh = plsc.VectorSubcoreMesh(core_axis_name="c", subcore_axis_name="s")
    return functools.partial(
        pl.kernel,
        functools.partial(_sc_gather_body, NC, K0C, K1C, CH, HT, NBUF,
                          DEPTH),
        out_type=jax.ShapeDtypeStruct((n_pad, 2 * HT), jnp.float32),
        mesh=mesh,
        scratch_types=(
            [pltpu.VMEM((kmax * CH,), jnp.int32),
             pltpu.VMEM((kmax * CH,), jnp.int32)] +
            [pltpu.VMEM((NBUF, CH, HT), jnp.float32) for _ in range(2)] +
            [pltpu.SemaphoreType.DMA for _ in range(2 * NBUF)]
        ),
    )()


def _unpack(x):
    w = lax.bitcast_convert_type(x, jnp.uint32)
    lo = lax.bitcast_convert_type(w << jnp.uint32(16), jnp.float32)
    hi = lax.bitcast_convert_type(w & jnp.uint32(0xFFFF0000), jnp.float32)
    return lo, hi


def _tc_body(HH, has_prev, *refs):
    if has_prev:
        (mail_ref, wft_ref, bf_ref, wiout_ref, biou_ref,
         _hprev, _cprev, hnew_ref, cnew_ref) = refs
    else:
        (mail_ref, wft_ref, bf_ref, wiout_ref, biou_ref,
         hnew_ref, cnew_ref) = refs
    HP = HH // 2
    mb = mail_ref[...]
    hlo0, hhi0 = _unpack(mb[:, 0 * HP:1 * HP])
    clo0, chi0 = _unpack(mb[:, 1 * HP:2 * HP])
    hlo1, hhi1 = _unpack(mb[:, 2 * HP:3 * HP])
    clo1, chi1 = _unpack(mb[:, 3 * HP:4 * HP])
    hcat = jnp.concatenate([hlo0, hhi0, hlo1, hhi1],
                           axis=1).astype(jnp.bfloat16)
    f_pre = jnp.dot(hcat, wft_ref[...],
                    preferred_element_type=jnp.float32) + bf_ref[...]
    f = jax.nn.sigmoid(f_pre)
    cc = jnp.concatenate([clo0, chi0, clo1, chi1], axis=1)
    c_red = f[:, :HH] * cc[:, :HH] + f[:, HH:] * cc[:, HH:]
    iou = jnp.dot(hcat, wiout_ref[...],
                  preferred_element_type=jnp.float32) + biou_ref[...]
    i = jax.nn.sigmoid(iou[:, :HH])
    o = jax.nn.sigmoid(iou[:, HH:2 * HH])
    u = jnp.tanh(iou[:, 2 * HH:])
    c_new = i * u + c_red
    hnew_ref[...] = o * jnp.tanh(c_new)
    cnew_ref[...] = c_new


def _tc_compute_seg(n, blk0, nblk, mail, wft, bf, wiout, biou, prev, M):
    fourHP = mail.shape[1]
    HH = fourHP // 2
    twoH = 2 * HH
    in_specs = [
        pl.BlockSpec((M, fourHP), lambda i: (i, 0)),
        pl.BlockSpec((twoH, twoH), lambda i: (0, 0)),
        pl.BlockSpec((1, twoH), lambda i: (0, 0)),
        pl.BlockSpec((twoH, 3 * HH), lambda i: (0, 0)),
        pl.BlockSpec((1, 3 * HH), lambda i: (0, 0)),
    ]
    args = [mail, wft, bf, wiout, biou]
    kwargs = {}
    if prev is not None:
        in_specs += [pl.BlockSpec(memory_space=pl.ANY),
                     pl.BlockSpec(memory_space=pl.ANY)]
        args += [prev[0], prev[1]]
        kwargs['input_output_aliases'] = {5: 0, 6: 1}
    return pl.pallas_call(
        functools.partial(_tc_body, HH, prev is not None),
        grid=(nblk,),
        in_specs=in_specs,
        out_specs=[
            pl.BlockSpec((M, HH), lambda i: (i + blk0, 0)),
            pl.BlockSpec((M, HH), lambda i: (i + blk0, 0)),
        ],
        out_shape=[
            jax.ShapeDtypeStruct((n, HH), jnp.float32),
            jax.ShapeDtypeStruct((n, HH), jnp.float32),
        ],
        **kwargs,
    )(*args)


def kernel(h, c, child_idx, W_f, b_f, W_iou, b_iou):
    n, HH = h.shape
    HT = HH
    NW = 32
    n_per_w = -(-n // (NW * 160)) * 160
    n_pad = n_per_w * NW

    ci = child_idx.astype(jnp.int32)
    pad = jnp.zeros((n_pad - n,), jnp.int32)
    idx0 = jnp.concatenate([ci[:, 0], pad])
    idx1 = jnp.concatenate([ci[:, 1], pad])

    table = _tc_pack(h, c)

    S = 2
    seg = n_pad // S
    M = 1600
    nblk = seg // M
    fracs = [0.5] + [0.85] * (S - 1)
    wft = W_f.T.astype(jnp.bfloat16)
    wiout = W_iou.T.astype(jnp.bfloat16)
    bf2 = b_f.reshape(1, -1)
    biou2 = b_iou.reshape(1, -1)

    prev = None
    for s in range(S):
        sc_gather = _make_sc_gather(seg, HT, frac0=fracs[s])
        mail = sc_gather(table, idx0[s * seg:(s + 1) * seg],
                         idx1[s * seg:(s + 1) * seg])
        prev = _tc_compute_seg(n, s * nblk, nblk, mail, wft, bf2, wiout,
                               biou2, prev, M)
    return (prev[0], prev[1])


# speedup vs baseline: 1.1520x; 1.0405x over previous
import functools

import jax
import jax.numpy as jnp
from jax import lax
from jax.experimental import pallas as pl
from jax.experimental.pallas import tpu as pltpu
from jax.experimental.pallas import tpu_sc as plsc


def _rne16(u):
    return (u + jnp.uint32(0x7FFF) + ((u >> jnp.uint32(16)) & jnp.uint32(1))
            ) >> jnp.uint32(16)


def _pack_pair(x, HP):
    a = lax.bitcast_convert_type(x[:, :HP], jnp.uint32)
    b = lax.bitcast_convert_type(x[:, HP:], jnp.uint32)
    w = (_rne16(a) & jnp.uint32(0xFFFF)) | (_rne16(b) << jnp.uint32(16))
    return lax.bitcast_convert_type(w, jnp.float32)


def _pack_body(HP, h_ref, c_ref, t_ref):
    t_ref[:, :HP] = _pack_pair(h_ref[...], HP)
    t_ref[:, HP:] = _pack_pair(c_ref[...], HP)


def _tc_pack(h, c, Mp=2000):
    n, HH = h.shape
    HP = HH // 2
    return pl.pallas_call(
        functools.partial(_pack_body, HP),
        grid=(n // Mp,),
        in_specs=[
            pl.BlockSpec((Mp, HH), lambda i: (i, 0)),
            pl.BlockSpec((Mp, HH), lambda i: (i, 0)),
        ],
        out_specs=pl.BlockSpec((Mp, 2 * HP), lambda i: (i, 0)),
        out_shape=jax.ShapeDtypeStruct((n, 2 * HP), jnp.float32),
    )(h, c)


def _sc_gather_body(NC, K0C, K1C, CH, HT, NBUF, DEPTH,
                    t_hbm, idx0_hbm, idx1_hbm, out,
                    i0_all, i1_all, g0_v, g1_v, *sems):
    gsems = sems[:NBUF]
    wsems = sems[NBUF:]
    cid = lax.axis_index("c")
    sid = lax.axis_index("s")
    start = jnp.where(cid == 0, sid * K0C, (16 * K0C) + sid * K1C)
    nch_w = jnp.where(cid == 0, K0C, K1C)
    wbase = start * CH

    @pl.when(cid == 0)
    def _():
        pltpu.sync_copy(idx0_hbm.at[pl.ds(wbase, K0C * CH)],
                        i0_all.at[pl.ds(0, K0C * CH)])
        pltpu.sync_copy(idx1_hbm.at[pl.ds(wbase, K0C * CH)],
                        i1_all.at[pl.ds(0, K0C * CH)])

    if K1C:
        @pl.when(cid == 1)
        def _():
            pltpu.sync_copy(idx0_hbm.at[pl.ds(wbase, K1C * CH)],
                            i0_all.at[pl.ds(0, K1C * CH)])
            pltpu.sync_copy(idx1_hbm.at[pl.ds(wbase, K1C * CH)],
                            i1_all.at[pl.ds(0, K1C * CH)])

    def g_cps(b, k):
        io0 = i0_all.at[pl.ds(k * CH, CH)]
        io1 = i1_all.at[pl.ds(k * CH, CH)]
        return (
            pltpu.make_async_copy(t_hbm.at[io0], g0_v.at[b], gsems[b]),
            pltpu.make_async_copy(t_hbm.at[io1], g1_v.at[b], gsems[b]),
        )

    def w_cps(b, k):
        rows = pl.ds(wbase + k * CH, CH)
        return (
            pltpu.make_async_copy(g0_v.at[b], out.at[rows, pl.ds(0, HT)],
                                  wsems[b]),
            pltpu.make_async_copy(g1_v.at[b], out.at[rows, pl.ds(HT, HT)],
                                  wsems[b]),
        )

    def body(it, carry):
        g = it * NBUF
        for b in range(NBUF):
            k = g + b

            @pl.when((k >= NBUF) & (k - NBUF < nch_w))
            def _():
                for cp in w_cps(b, k - NBUF):
                    cp.wait()

            @pl.when(k < nch_w)
            def _():
                for cp in g_cps(b, k):
                    cp.start()

            kd = k - DEPTH
            bd = (b - DEPTH) % NBUF

            @pl.when((kd >= 0) & (kd < nch_w))
            def _():
                for cp in g_cps(bd, kd):
                    cp.wait()
                for cp in w_cps(bd, kd):
                    cp.start()
        return carry

    kmax = max(K0C, K1C)
    lax.fori_loop(0, (kmax + 2 * NBUF - 1) // NBUF, body, 0)


def _make_sc_gather(n_pad, HT, frac0=0.5):
    info = plsc.get_sparse_core_info()
    NC, NS = info.num_cores, info.num_subcores
    NW = NC * NS
    CH = 40
    NBUF = 4
    DEPTH = 2
    assert n_pad % (NW * CH) == 0
    nch_g = n_pad // CH
    K0C = int(round(nch_g * frac0 / NS))
    K1C = nch_g // NS - K0C
    kmax = max(K0C, K1C)

    mesh = plsc.VectorSubcoreMesh(core_axis_name="c", subcore_axis_name="s")
    return functools.partial(
        pl.kernel,
        functools.partial(_sc_gather_body, NC, K0C, K1C, CH, HT, NBUF,
                          DEPTH),
        out_type=jax.ShapeDtypeStruct((n_pad, 2 * HT), jnp.float32),
        mesh=mesh,
        scratch_types=(
            [pltpu.VMEM((kmax * CH,), jnp.int32),
             pltpu.VMEM((kmax * CH,), jnp.int32)] +
            [pltpu.VMEM((NBUF, CH, HT), jnp.float32) for _ in range(2)] +
            [pltpu.SemaphoreType.DMA for _ in range(2 * NBUF)]
        ),
    )()


def _unpack(x):
    w = lax.bitcast_convert_type(x, jnp.uint32)
    lo = lax.bitcast_convert_type(w << jnp.uint32(16), jnp.float32)
    hi = lax.bitcast_convert_type(w & jnp.uint32(0xFFFF0000), jnp.float32)
    return lo, hi


def _tc_body(HH, has_prev, *refs):
    if has_prev:
        (mail_ref, wft_ref, bf_ref, wiout_ref, biou_ref,
         _hprev, _cprev, hnew_ref, cnew_ref) = refs
    else:
        (mail_ref, wft_ref, bf_ref, wiout_ref, biou_ref,
         hnew_ref, cnew_ref) = refs
    HP = HH // 2
    mb = mail_ref[...]
    hlo0, hhi0 = _unpack(mb[:, 0 * HP:1 * HP])
    clo0, chi0 = _unpack(mb[:, 1 * HP:2 * HP])
    hlo1, hhi1 = _unpack(mb[:, 2 * HP:3 * HP])
    clo1, chi1 = _unpack(mb[:, 3 * HP:4 * HP])
    hcat = jnp.concatenate([hlo0, hhi0, hlo1, hhi1],
                           axis=1).astype(jnp.bfloat16)
    f_pre = jnp.dot(hcat, wft_ref[...],
                    preferred_element_type=jnp.float32) + bf_ref[...]
    f = jax.nn.sigmoid(f_pre)
    cc = jnp.concatenate([clo0, chi0, clo1, chi1], axis=1)
    c_red = f[:, :HH] * cc[:, :HH] + f[:, HH:] * cc[:, HH:]
    iou = jnp.dot(hcat, wiout_ref[...],
                  preferred_element_type=jnp.float32) + biou_ref[...]
    i = jax.nn.sigmoid(iou[:, :HH])
    o = jax.nn.sigmoid(iou[:, HH:2 * HH])
    u = jnp.tanh(iou[:, 2 * HH:])
    c_new = i * u + c_red
    hnew_ref[...] = o * jnp.tanh(c_new)
    cnew_ref[...] = c_new


def _tc_compute_seg(n, blk0, nblk, mail, wft, bf, wiout, biou, prev, M):
    fourHP = mail.shape[1]
    HH = fourHP // 2
    twoH = 2 * HH
    in_specs = [
        pl.BlockSpec((M, fourHP), lambda i: (i, 0)),
        pl.BlockSpec((twoH, twoH), lambda i: (0, 0)),
        pl.BlockSpec((1, twoH), lambda i: (0, 0)),
        pl.BlockSpec((twoH, 3 * HH), lambda i: (0, 0)),
        pl.BlockSpec((1, 3 * HH), lambda i: (0, 0)),
    ]
    args = [mail, wft, bf, wiout, biou]
    kwargs = {}
    if prev is not None:
        in_specs += [pl.BlockSpec(memory_space=pl.ANY),
                     pl.BlockSpec(memory_space=pl.ANY)]
        args += [prev[0], prev[1]]
        kwargs['input_output_aliases'] = {5: 0, 6: 1}
    return pl.pallas_call(
        functools.partial(_tc_body, HH, prev is not None),
        grid=(nblk,),
        in_specs=in_specs,
        out_specs=[
            pl.BlockSpec((M, HH), lambda i: (i + blk0, 0)),
            pl.BlockSpec((M, HH), lambda i: (i + blk0, 0)),
        ],
        out_shape=[
            jax.ShapeDtypeStruct((n, HH), jnp.float32),
            jax.ShapeDtypeStruct((n, HH), jnp.float32),
        ],
        **kwargs,
    )(*args)


def kernel(h, c, child_idx, W_f, b_f, W_iou, b_iou):
    n, HH = h.shape
    HT = HH
    NW = 32
    n_per_w = -(-n // (NW * 160)) * 160
    n_pad = n_per_w * NW

    ci = child_idx.astype(jnp.int32)
    pad = jnp.zeros((n_pad - n,), jnp.int32)
    idx0 = jnp.concatenate([ci[:, 0], pad])
    idx1 = jnp.concatenate([ci[:, 1], pad])

    table = _tc_pack(h, c)

    M = 1600
    seg_nodes = [38400, 12800]
    fracs = [0.5, 1.0]
    wft = W_f.T.astype(jnp.bfloat16)
    wiout = W_iou.T.astype(jnp.bfloat16)
    bf2 = b_f.reshape(1, -1)
    biou2 = b_iou.reshape(1, -1)

    prev = None
    off = 0
    blk0 = 0
    for seg, frac in zip(seg_nodes, fracs):
        sc_gather = _make_sc_gather(seg, HT, frac0=frac)
        mail = sc_gather(table, idx0[off:off + seg],
                         idx1[off:off + seg])
        prev = _tc_compute_seg(n, blk0, seg // M, mail, wft, bf2, wiout,
                               biou2, prev, M)
        off += seg
        blk0 += seg // M
    return (prev[0], prev[1])


# speedup vs baseline: 1.1533x; 1.0011x over previous
import functools

import jax
import jax.numpy as jnp
from jax import lax
from jax.experimental import pallas as pl
from jax.experimental.pallas import tpu as pltpu
from jax.experimental.pallas import tpu_sc as plsc


def _rne16(u):
    return (u + jnp.uint32(0x7FFF) + ((u >> jnp.uint32(16)) & jnp.uint32(1))
            ) >> jnp.uint32(16)


def _pack_pair(x, HP):
    a = lax.bitcast_convert_type(x[:, :HP], jnp.uint32)
    b = lax.bitcast_convert_type(x[:, HP:], jnp.uint32)
    w = (_rne16(a) & jnp.uint32(0xFFFF)) | (_rne16(b) << jnp.uint32(16))
    return lax.bitcast_convert_type(w, jnp.float32)


def _pack_body(HP, h_ref, c_ref, t_ref):
    t_ref[:, :HP] = _pack_pair(h_ref[...], HP)
    t_ref[:, HP:] = _pack_pair(c_ref[...], HP)


def _tc_pack(h, c, Mp=2000):
    n, HH = h.shape
    HP = HH // 2
    return pl.pallas_call(
        functools.partial(_pack_body, HP),
        grid=(n // Mp,),
        in_specs=[
            pl.BlockSpec((Mp, HH), lambda i: (i, 0)),
            pl.BlockSpec((Mp, HH), lambda i: (i, 0)),
        ],
        out_specs=pl.BlockSpec((Mp, 2 * HP), lambda i: (i, 0)),
        out_shape=jax.ShapeDtypeStruct((n, 2 * HP), jnp.float32),
    )(h, c)


def _sc_gather_body(NC, K0C, K1C, CH, HT, NBUF, DEPTH,
                    t_hbm, idx0_hbm, idx1_hbm, out,
                    i0_all, i1_all, g0_v, g1_v, *sems):
    gsems = sems[:NBUF]
    wsems = sems[NBUF:]
    cid = lax.axis_index("c")
    sid = lax.axis_index("s")
    start = jnp.where(cid == 0, sid * K0C, (16 * K0C) + sid * K1C)
    nch_w = jnp.where(cid == 0, K0C, K1C)
    wbase = start * CH

    @pl.when(cid == 0)
    def _():
        pltpu.sync_copy(idx0_hbm.at[pl.ds(wbase, K0C * CH)],
                        i0_all.at[pl.ds(0, K0C * CH)])
        pltpu.sync_copy(idx1_hbm.at[pl.ds(wbase, K0C * CH)],
                        i1_all.at[pl.ds(0, K0C * CH)])

    if K1C:
        @pl.when(cid == 1)
        def _():
            pltpu.sync_copy(idx0_hbm.at[pl.ds(wbase, K1C * CH)],
                            i0_all.at[pl.ds(0, K1C * CH)])
            pltpu.sync_copy(idx1_hbm.at[pl.ds(wbase, K1C * CH)],
                            i1_all.at[pl.ds(0, K1C * CH)])

    def g_cps(b, k):
        io0 = i0_all.at[pl.ds(k * CH, CH)]
        io1 = i1_all.at[pl.ds(k * CH, CH)]
        return (
            pltpu.make_async_copy(t_hbm.at[io0], g0_v.at[b], gsems[b]),
            pltpu.make_async_copy(t_hbm.at[io1], g1_v.at[b], gsems[b]),
        )

    def w_cps(b, k):
        rows = pl.ds(wbase + k * CH, CH)
        return (
            pltpu.make_async_copy(g0_v.at[b], out.at[rows, pl.ds(0, HT)],
                                  wsems[b]),
            pltpu.make_async_copy(g1_v.at[b], out.at[rows, pl.ds(HT, HT)],
                                  wsems[b]),
        )

    def body(it, carry):
        g = it * NBUF
        for b in range(NBUF):
            k = g + b

            @pl.when((k >= NBUF) & (k - NBUF < nch_w))
            def _():
                for cp in w_cps(b, k - NBUF):
                    cp.wait()

            @pl.when(k < nch_w)
            def _():
                for cp in g_cps(b, k):
                    cp.start()

            kd = k - DEPTH
            bd = (b - DEPTH) % NBUF

            @pl.when((kd >= 0) & (kd < nch_w))
            def _():
                for cp in g_cps(bd, kd):
                    cp.wait()
                for cp in w_cps(bd, kd):
                    cp.start()
        return carry

    kmax = max(K0C, K1C)
    lax.fori_loop(0, (kmax + 2 * NBUF - 1) // NBUF, body, 0)


def _make_sc_gather(n_pad, HT, frac0=0.5):
    info = plsc.get_sparse_core_info()
    NC, NS = info.num_cores, info.num_subcores
    NW = NC * NS
    CH = 40
    NBUF = 4
    DEPTH = 2
    assert n_pad % (NW * CH) == 0
    nch_g = n_pad // CH
    K0C = int(round(nch_g * frac0 / NS))
    K1C = nch_g // NS - K0C
    kmax = max(K0C, K1C)

    mesh = plsc.VectorSubcoreMesh(core_axis_name="c", subcore_axis_name="s")
    return functools.partial(
        pl.kernel,
        functools.partial(_sc_gather_body, NC, K0C, K1C, CH, HT, NBUF,
                          DEPTH),
        out_type=jax.ShapeDtypeStruct((n_pad, 2 * HT), jnp.float32),
        mesh=mesh,
        scratch_types=(
            [pltpu.VMEM((kmax * CH,), jnp.int32),
             pltpu.VMEM((kmax * CH,), jnp.int32)] +
            [pltpu.VMEM((NBUF, CH, HT), jnp.float32) for _ in range(2)] +
            [pltpu.SemaphoreType.DMA for _ in range(2 * NBUF)]
        ),
    )()


def _unpack(x):
    w = lax.bitcast_convert_type(x, jnp.uint32)
    lo = lax.bitcast_convert_type(w << jnp.uint32(16), jnp.float32)
    hi = lax.bitcast_convert_type(w & jnp.uint32(0xFFFF0000), jnp.float32)
    return lo, hi


def _tc_body(HH, has_prev, *refs):
    if has_prev:
        (mail_ref, wft_ref, bf_ref, wiout_ref, biou_ref,
         _hprev, _cprev, hnew_ref, cnew_ref) = refs
    else:
        (mail_ref, wft_ref, bf_ref, wiout_ref, biou_ref,
         hnew_ref, cnew_ref) = refs
    HP = HH // 2
    mb = mail_ref[...]
    hlo0, hhi0 = _unpack(mb[:, 0 * HP:1 * HP])
    clo0, chi0 = _unpack(mb[:, 1 * HP:2 * HP])
    hlo1, hhi1 = _unpack(mb[:, 2 * HP:3 * HP])
    clo1, chi1 = _unpack(mb[:, 3 * HP:4 * HP])
    hcat = jnp.concatenate([hlo0, hhi0, hlo1, hhi1],
                           axis=1).astype(jnp.bfloat16)
    f_pre = jnp.dot(hcat, wft_ref[...],
                    preferred_element_type=jnp.float32) + bf_ref[...]
    f = jax.nn.sigmoid(f_pre)
    cc = jnp.concatenate([clo0, chi0, clo1, chi1], axis=1)
    c_red = f[:, :HH] * cc[:, :HH] + f[:, HH:] * cc[:, HH:]
    iou = jnp.dot(hcat, wiout_ref[...],
                  preferred_element_type=jnp.float32) + biou_ref[...]
    i = jax.nn.sigmoid(iou[:, :HH])
    o = jax.nn.sigmoid(iou[:, HH:2 * HH])
    u = jnp.tanh(iou[:, 2 * HH:])
    c_new = i * u + c_red
    hnew_ref[...] = o * jnp.tanh(c_new)
    cnew_ref[...] = c_new


def _tc_compute_seg(n, blk0, nblk, mail, wft, bf, wiout, biou, prev, M):
    fourHP = mail.shape[1]
    HH = fourHP // 2
    twoH = 2 * HH
    in_specs = [
        pl.BlockSpec((M, fourHP), lambda i: (i, 0)),
        pl.BlockSpec((twoH, twoH), lambda i: (0, 0)),
        pl.BlockSpec((1, twoH), lambda i: (0, 0)),
        pl.BlockSpec((twoH, 3 * HH), lambda i: (0, 0)),
        pl.BlockSpec((1, 3 * HH), lambda i: (0, 0)),
    ]
    args = [mail, wft, bf, wiout, biou]
    kwargs = {}
    if prev is not None:
        in_specs += [pl.BlockSpec(memory_space=pl.ANY),
                     pl.BlockSpec(memory_space=pl.ANY)]
        args += [prev[0], prev[1]]
        kwargs['input_output_aliases'] = {5: 0, 6: 1}
    return pl.pallas_call(
        functools.partial(_tc_body, HH, prev is not None),
        grid=(nblk,),
        in_specs=in_specs,
        out_specs=[
            pl.BlockSpec((M, HH), lambda i: (i + blk0, 0)),
            pl.BlockSpec((M, HH), lambda i: (i + blk0, 0)),
        ],
        out_shape=[
            jax.ShapeDtypeStruct((n, HH), jnp.float32),
            jax.ShapeDtypeStruct((n, HH), jnp.float32),
        ],
        **kwargs,
    )(*args)


def kernel(h, c, child_idx, W_f, b_f, W_iou, b_iou):
    n, HH = h.shape
    HT = HH
    NW = 32
    n_per_w = -(-n // (NW * 160)) * 160
    n_pad = n_per_w * NW

    ci = child_idx.astype(jnp.int32)
    pad = jnp.zeros((n_pad - n,), jnp.int32)
    idx0 = jnp.concatenate([ci[:, 0], pad])
    idx1 = jnp.concatenate([ci[:, 1], pad])

    table = _tc_pack(h, c)

    M = 1600
    seg_nodes = [38400, 12800]
    fracs = [0.5, 0.5]
    wft = W_f.T.astype(jnp.bfloat16)
    wiout = W_iou.T.astype(jnp.bfloat16)
    bf2 = b_f.reshape(1, -1)
    biou2 = b_iou.reshape(1, -1)

    prev = None
    off = 0
    blk0 = 0
    for seg, frac in zip(seg_nodes, fracs):
        sc_gather = _make_sc_gather(seg, HT, frac0=frac)
        mail = sc_gather(table, idx0[off:off + seg],
                         idx1[off:off + seg])
        prev = _tc_compute_seg(n, blk0, seg // M, mail, wft, bf2, wiout,
                               biou2, prev, M)
        off += seg
        blk0 += seg // M
    return (prev[0], prev[1])
